# baseline (speedup 1.0000x reference)
import jax
import jax.numpy as jnp
from jax.experimental import pallas as pl


_ROWS = 1024


def _body(x_ref, y_ref, o_ref):
    xb = x_ref[...]
    o_ref[...] = (
        jnp.concatenate([xb[:, :, 3:4], xb[:, :, 1:2], xb[:, :, 2:3]], axis=-1)
        + y_ref[...]
    )


def kernel(x, y):
    B, S, D = x.shape
    J = y.shape[-1]
    grid = (B, S // _ROWS)
    return pl.pallas_call(
        _body,
        grid=grid,
        in_specs=[
            pl.BlockSpec((1, _ROWS, 128), lambda b, i: (b, i, 0)),
            pl.BlockSpec((1, _ROWS, J), lambda b, i: (b, i, 0)),
        ],
        out_specs=pl.BlockSpec((1, _ROWS, J), lambda b, i: (b, i, 0)),
        out_shape=jax.ShapeDtypeStruct((B, S, J), x.dtype),
    )(x, y)


# speedup vs baseline: 1.2776x; 1.2776x over previous
import jax
import jax.numpy as jnp
from jax.experimental import pallas as pl


_ROWS = 4096


def _body(x_ref, y_ref, o_ref):
    xb = x_ref[...]
    o_ref[...] = (
        jnp.concatenate([xb[:, :, 3:4], xb[:, :, 1:2], xb[:, :, 2:3]], axis=-1)
        + y_ref[...]
    )


def kernel(x, y):
    B, S, D = x.shape
    J = y.shape[-1]
    grid = (B, S // _ROWS)
    return pl.pallas_call(
        _body,
        grid=grid,
        in_specs=[
            pl.BlockSpec((1, _ROWS, 128), lambda b, i: (b, i, 0)),
            pl.BlockSpec((1, _ROWS, J), lambda b, i: (b, i, 0)),
        ],
        out_specs=pl.BlockSpec((1, _ROWS, J), lambda b, i: (b, i, 0)),
        out_shape=jax.ShapeDtypeStruct((B, S, J), x.dtype),
    )(x, y)


# speedup vs baseline: 1.3223x; 1.0349x over previous
import jax
import jax.numpy as jnp
from jax.experimental import pallas as pl


_ROWS = 8192


def _body(x_ref, y_ref, o_ref):
    xb = x_ref[...]
    o_ref[...] = (
        jnp.concatenate([xb[:, :, 3:4], xb[:, :, 1:2], xb[:, :, 2:3]], axis=-1)
        + y_ref[...]
    )


def kernel(x, y):
    B, S, D = x.shape
    J = y.shape[-1]
    grid = (B, S // _ROWS)
    return pl.pallas_call(
        _body,
        grid=grid,
        in_specs=[
            pl.BlockSpec((1, _ROWS, 128), lambda b, i: (b, i, 0)),
            pl.BlockSpec((1, _ROWS, J), lambda b, i: (b, i, 0)),
        ],
        out_specs=pl.BlockSpec((1, _ROWS, J), lambda b, i: (b, i, 0)),
        out_shape=jax.ShapeDtypeStruct((B, S, J), x.dtype),
    )(x, y)
